# Initial kernel scaffold; baseline (speedup 1.0000x reference)
import jax
import jax.numpy as jnp
from jax.experimental import pallas as pl


def kernel(adj_edge_index, adj_edge_values, uEmbeds, iEmbeds):
    raise NotImplementedError("write your pallas kernel here")


# speedup vs baseline: 4.0366x; 4.0366x over previous
import jax
import jax.numpy as jnp
from jax import lax
from jax.experimental import pallas as pl
from jax.experimental.pallas import tpu as pltpu
from jax.experimental.pallas import tpu_sc as plsc

USER_N = 5000
ITEM_N = 5000
N = USER_N + ITEM_N
D = 128
DH = 64
NLAYERS = 3
E = 320000
NC = 2
NS = 16
CHUNK = 128
NCH = 157
EPT = NCH * CHUNK
E_PAD = NS * EPT
NP = 10240
RPT = NP // NS
RC = 128
ZR = 32


def _sc_body(emb_hbm, src_hbm, dst_hbm, vals_hbm, out_hbm, cur_hbm,
             src_ref, dst_ref, vals_ref, gbuf, abuf, zbuf, b_sh):
    c_id = lax.axis_index("c")
    s_id = lax.axis_index("s")
    base = s_id * RPT
    cbase = c_id * NP + base

    pltpu.sync_copy(src_hbm.at[c_id, s_id], src_ref)
    pltpu.sync_copy(dst_hbm.at[s_id], dst_ref)
    pltpu.sync_copy(vals_hbm.at[s_id], vals_ref)

    pltpu.sync_copy(emb_hbm.at[pl.ds(cbase, RPT)], out_hbm.at[pl.ds(cbase, RPT)])
    pltpu.sync_copy(emb_hbm.at[pl.ds(cbase, RPT)], cur_hbm.at[pl.ds(cbase, RPT)])

    def _zb(i, carry):
        for q in range(4):
            zbuf[i, pl.ds(q * 16, 16)] = jnp.zeros((16,), jnp.float32)
        return carry
    lax.fori_loop(0, ZR, _zb, 0)
    for k in range(RPT // ZR):
        pltpu.sync_copy(zbuf, b_sh.at[pl.ds(base + k * ZR, ZR)])
    plsc.subcore_barrier()

    def _layer(l, carry):
        def _chunk(j, carry2):
            pltpu.sync_copy(cur_hbm.at[src_ref.at[j]], gbuf)
            for g in range(CHUNK // 16):
                vv = vals_ref[j, pl.ds(g * 16, 16)]
                for i in range(16):
                    e = g * 16 + i
                    v = vv[i]
                    for q in range(4):
                        sl = pl.ds(q * 16, 16)
                        gbuf[e, sl] = gbuf[e, sl] * v
            pltpu.sync_copy(gbuf, b_sh.at[dst_ref.at[j]], add=True)
            return carry2
        lax.fori_loop(0, NCH, _chunk, 0)
        plsc.subcore_barrier()

        for k in range(RPT // RC):
            rb = base + k * RC
            cb = cbase + k * RC
            pltpu.sync_copy(b_sh.at[pl.ds(rb, RC)], gbuf)
            for z in range(RC // ZR):
                pltpu.sync_copy(zbuf, b_sh.at[pl.ds(rb + z * ZR, ZR)])
            pltpu.sync_copy(out_hbm.at[pl.ds(cb, RC)], abuf)

            def _acc(i, carry3):
                for q in range(4):
                    sl = pl.ds(q * 16, 16)
                    abuf[i, sl] = abuf[i, sl] + gbuf[i, sl]
                return carry3
            lax.fori_loop(0, RC, _acc, 0)
            pltpu.sync_copy(abuf, out_hbm.at[pl.ds(cb, RC)])
            pltpu.sync_copy(gbuf, cur_hbm.at[pl.ds(cb, RC)])
        plsc.subcore_barrier()
        return carry

    lax.fori_loop(0, NLAYERS, _layer, 0)


def _make_call():
    mesh = plsc.VectorSubcoreMesh(core_axis_name="c", subcore_axis_name="s",
                                  num_cores=NC, num_subcores=NS)
    return pl.kernel(
        _sc_body,
        out_type=(
            jax.ShapeDtypeStruct((NC * NP, DH), jnp.float32),
            jax.ShapeDtypeStruct((NC * NP, DH), jnp.float32),
        ),
        mesh=mesh,
        compiler_params=pltpu.CompilerParams(use_tc_tiling_on_sc=False),
        scratch_types=[
            pltpu.VMEM((NCH, CHUNK), jnp.int32),
            pltpu.VMEM((NCH, CHUNK), jnp.int32),
            pltpu.VMEM((NCH, CHUNK), jnp.float32),
            pltpu.VMEM((CHUNK, DH), jnp.float32),
            pltpu.VMEM((RC, DH), jnp.float32),
            pltpu.VMEM((ZR, DH), jnp.float32),
            pltpu.VMEM_SHARED((NP, DH), jnp.float32),
        ],
    )


_sc_call = _make_call()


def kernel(adj_edge_index, adj_edge_values, uEmbeds, iEmbeds):
    embeds = jnp.concatenate([uEmbeds, iEmbeds], axis=0)
    rpad = jnp.zeros((NP - N, DH), jnp.float32)
    emb_flat = jnp.concatenate(
        [embeds[:, :DH], rpad, embeds[:, DH:], rpad], axis=0)

    dst = adj_edge_index[0]
    src = adj_edge_index[1]
    npad = E_PAD - E
    pad_idx = (jnp.arange(npad, dtype=jnp.int32) * 61) % N
    src_p = jnp.concatenate([src, pad_idx])
    dst_p = jnp.concatenate([dst, pad_idx])
    vals_p = jnp.concatenate([adj_edge_values,
                              jnp.zeros((npad,), jnp.float32)])

    src_a = jnp.stack([src_p, src_p + NP]).reshape(NC, NS, NCH, CHUNK)
    dst_a = dst_p.reshape(NS, NCH, CHUNK)
    vals_a = vals_p.reshape(NS, NCH, CHUNK)

    out_flat, _ = _sc_call(emb_flat, src_a, dst_a, vals_a)
    out = jnp.concatenate([out_flat[:N], out_flat[NP:NP + N]], axis=1)
    return (out[:USER_N], out[USER_N:])


# speedup vs baseline: 5.2463x; 1.2997x over previous
import jax
import jax.numpy as jnp
from jax import lax
from jax.experimental import pallas as pl
from jax.experimental.pallas import tpu as pltpu
from jax.experimental.pallas import tpu_sc as plsc

USER_N = 5000
ITEM_N = 5000
N = USER_N + ITEM_N
D = 128
DH = 64
NLAYERS = 3
E = 320000
NC = 2
NS = 16
CHUNK = 128
NCH = 160
EPT = NCH * CHUNK
E_PAD = NS * EPT
NP = 10240
RPT = NP // NS
RC = 128
ZR = 32
NB = 4
NEB = 8


def _sc_body(emb_hbm, edges_hbm, vals_hbm, out_hbm, cur_hbm,
             ebuf, vbuf, gbuf, abuf, zbuf, b_sh, esem, gsem, ssem):
    c_id = lax.axis_index("c")
    s_id = lax.axis_index("s")
    base = s_id * RPT
    cbase = c_id * NP + base

    def issue_e(j, slot):
        pltpu.async_copy(edges_hbm.at[c_id, s_id, j], ebuf.at[slot],
                         esem.at[slot])
        pltpu.async_copy(vals_hbm.at[s_id, j], vbuf.at[slot], esem.at[slot])

    def wait_e(j, slot):
        pltpu.make_async_copy(edges_hbm.at[c_id, s_id, j], ebuf.at[slot],
                              esem.at[slot]).wait()
        pltpu.make_async_copy(vals_hbm.at[s_id, j], vbuf.at[slot],
                              esem.at[slot]).wait()

    def issue_g(slot, b):
        pltpu.async_copy(cur_hbm.at[ebuf.at[slot, 0]], gbuf.at[b],
                         gsem.at[b])

    def wait_g(slot, b):
        pltpu.make_async_copy(cur_hbm.at[ebuf.at[slot, 0]], gbuf.at[b],
                              gsem.at[b]).wait()

    def issue_s(slot, b):
        pltpu.async_copy(gbuf.at[b], b_sh.at[ebuf.at[slot, 1]],
                         ssem.at[b], add=True)

    def wait_s(slot, b):
        pltpu.make_async_copy(gbuf.at[b], b_sh.at[ebuf.at[slot, 1]],
                              ssem.at[b]).wait()

    pltpu.sync_copy(emb_hbm.at[pl.ds(cbase, RPT)], out_hbm.at[pl.ds(cbase, RPT)])
    pltpu.sync_copy(emb_hbm.at[pl.ds(cbase, RPT)], cur_hbm.at[pl.ds(cbase, RPT)])

    def _zb(i, carry):
        for q in range(4):
            zbuf[i, pl.ds(q * 16, 16)] = jnp.zeros((16,), jnp.float32)
        return carry
    lax.fori_loop(0, ZR, _zb, 0)
    for k in range(RPT // ZR):
        pltpu.sync_copy(zbuf, b_sh.at[pl.ds(base + k * ZR, ZR)])
    plsc.subcore_barrier()

    def _layer(l, carry):
        for j in range(NB):
            issue_e(j, j)
        for j in range(2):
            wait_e(j, j)
            issue_g(j, j)

        def _quad(g4, carry2):
            j0 = g4 * NB
            for b in range(NB):
                j = j0 + b
                slot = lax.rem(j, NEB)
                wait_g(slot, b)
                @pl.when(j + NB < NCH)
                def _():
                    issue_e(j + NB, lax.rem(j + NB, NEB))
                for grp in range(CHUNK // 16):
                    vv = vbuf[slot, pl.ds(grp * 16, 16)]
                    for i in range(16):
                        e = grp * 16 + i
                        v = vv[i]
                        for q in range(4):
                            sl = pl.ds(q * 16, 16)
                            gbuf[b, e, sl] = gbuf[b, e, sl] * v
                issue_s(slot, b)
                b2 = (b + 2) % NB
                @pl.when(j >= 2)
                def _():
                    wait_s(lax.rem(j - 2, NEB), b2)

                @pl.when(j + 2 < NCH)
                def _():
                    slot2 = lax.rem(j + 2, NEB)
                    wait_e(j + 2, slot2)
                    issue_g(slot2, b2)
            return carry2
        lax.fori_loop(0, NCH // NB, _quad, 0)
        wait_s(lax.rem(NCH - 2, NEB), (NCH - 2) % NB)
        wait_s(lax.rem(NCH - 1, NEB), (NCH - 1) % NB)
        plsc.subcore_barrier()

        for k in range(RPT // RC):
            rb = base + k * RC
            cb = cbase + k * RC
            pltpu.sync_copy(b_sh.at[pl.ds(rb, RC)], gbuf.at[0])
            for z in range(RC // ZR):
                pltpu.sync_copy(zbuf, b_sh.at[pl.ds(rb + z * ZR, ZR)])
            pltpu.sync_copy(out_hbm.at[pl.ds(cb, RC)], abuf)

            def _acc(i, carry3):
                for q in range(4):
                    sl = pl.ds(q * 16, 16)
                    abuf[i, sl] = abuf[i, sl] + gbuf[0, i, sl]
                return carry3
            lax.fori_loop(0, RC, _acc, 0)
            pltpu.sync_copy(abuf, out_hbm.at[pl.ds(cb, RC)])
            pltpu.sync_copy(gbuf.at[0], cur_hbm.at[pl.ds(cb, RC)])
        plsc.subcore_barrier()
        return carry

    lax.fori_loop(0, NLAYERS, _layer, 0)


def _make_call():
    mesh = plsc.VectorSubcoreMesh(core_axis_name="c", subcore_axis_name="s",
                                  num_cores=NC, num_subcores=NS)
    return pl.kernel(
        _sc_body,
        out_type=(
            jax.ShapeDtypeStruct((NC * NP, DH), jnp.float32),
            jax.ShapeDtypeStruct((NC * NP, DH), jnp.float32),
        ),
        mesh=mesh,
        compiler_params=pltpu.CompilerParams(use_tc_tiling_on_sc=False),
        scratch_types=[
            pltpu.VMEM((NEB, 2, CHUNK), jnp.int32),
            pltpu.VMEM((NEB, CHUNK), jnp.float32),
            pltpu.VMEM((NB, CHUNK, DH), jnp.float32),
            pltpu.VMEM((RC, DH), jnp.float32),
            pltpu.VMEM((ZR, DH), jnp.float32),
            pltpu.VMEM_SHARED((NP, DH), jnp.float32),
            pltpu.SemaphoreType.DMA((NEB,)),
            pltpu.SemaphoreType.DMA((NB,)),
            pltpu.SemaphoreType.DMA((NB,)),
        ],
    )


_sc_call = _make_call()


def kernel(adj_edge_index, adj_edge_values, uEmbeds, iEmbeds):
    embeds = jnp.concatenate([uEmbeds, iEmbeds], axis=0)
    rpad = jnp.zeros((NP - N, DH), jnp.float32)
    emb_flat = jnp.concatenate(
        [embeds[:, :DH], rpad, embeds[:, DH:], rpad], axis=0)

    dst = adj_edge_index[0]
    src = adj_edge_index[1]
    npad = E_PAD - E
    pad_idx = (jnp.arange(npad, dtype=jnp.int32) * 61) % N
    src_p = jnp.concatenate([src, pad_idx])
    dst_p = jnp.concatenate([dst, pad_idx])
    vals_p = jnp.concatenate([adj_edge_values,
                              jnp.zeros((npad,), jnp.float32)])

    edges = jnp.stack([
        jnp.stack([src_p, dst_p]),
        jnp.stack([src_p + NP, dst_p]),
    ])
    edges_a = edges.reshape(NC, 2, NS, NCH, CHUNK).transpose(0, 2, 3, 1, 4)
    vals_a = vals_p.reshape(NS, NCH, CHUNK)

    out_flat, _ = _sc_call(emb_flat, edges_a, vals_a)
    out = jnp.concatenate([out_flat[:N], out_flat[NP:NP + N]], axis=1)
    return (out[:USER_N], out[USER_N:])
